# baseline (speedup 1.0000x reference)
import functools

import jax
import jax.numpy as jnp
from jax import lax
from jax.experimental import pallas as pl
from jax.experimental.pallas import tpu as pltpu
from jax.experimental.pallas import tpu_sc as plsc

NPOINT = 512
NSAMPLE = 64
EPS = 1e-5
B = 2
N = 16384
C = 16
NROW = B * NPOINT
BLK = 4096
BMW = 32
NBM = N // BMW
CAP = 256
NW = 32
ROWS_PW = NROW // NW
NCH = 3 + C
NJOB = B * NCH


def _bn_eval(x, g, b, m, v):
    return (x - m[None, :, None]) / jnp.sqrt(v[None, :, None] + EPS) \
        * g[None, :, None] + b[None, :, None]


def _scores(xyz, W1, g1, b1, m1, v1, W2, g2, b2, m2, v2, W3, bias3):
    xyz_trans = jnp.transpose(xyz, (0, 2, 1))
    h = jnp.einsum('oi,bin->bon', W1, xyz_trans)
    h = jax.nn.relu(_bn_eval(h, g1, b1, m1, v1))
    h = jnp.einsum('oi,bin->bon', W2, h)
    h = jax.nn.relu(_bn_eval(h, g2, b2, m2, v2))
    logits = jnp.einsum('oi,bin->bon', W3, h) + bias3[None, :, None]
    return xyz_trans, jax.nn.sigmoid(logits)


def _bm_body(p_ref, bm_ref):
    bm_ref[0, 0] = jnp.max(
        p_ref[0].reshape(NPOINT, BLK // BMW, BMW), axis=-1)


def _block_max(p):
    nblk = N // BLK
    bm = pl.pallas_call(
        _bm_body,
        grid=(B, nblk),
        in_specs=[pl.BlockSpec((1, NPOINT, BLK), lambda b, j: (b, 0, j))],
        out_specs=pl.BlockSpec((1, 1, NPOINT, BLK // BMW),
                               lambda b, j: (b, j, 0, 0)),
        out_shape=jax.ShapeDtypeStruct((B, nblk, NPOINT, BLK // BMW),
                                       jnp.float32),
        compiler_params=pltpu.CompilerParams(
            dimension_semantics=("parallel", "parallel")),
    )(p)
    return bm.transpose(0, 2, 1, 3).reshape(NROW, NBM)


def _make_collect():
    mesh = plsc.VectorSubcoreMesh(core_axis_name="c", subcore_axis_name="s")
    QB = NSAMPLE + 16

    @functools.partial(
        pl.kernel,
        out_type=[
            jax.ShapeDtypeStruct((NROW, CAP), jnp.float32),
            jax.ShapeDtypeStruct((NROW, CAP), jnp.int32),
        ],
        mesh=mesh,
        compiler_params=pltpu.CompilerParams(needs_layout_passes=False),
    scratch_types=[
            pltpu.VMEM((NBM,), jnp.float32),
            pltpu.VMEM((QB + 16,), jnp.int32),
            pltpu.VMEM((QB + 16,), jnp.int32),
            pltpu.VMEM((QB, 128), jnp.float32),
            pltpu.VMEM((ROWS_PW * 16,), jnp.float32),
            pltpu.VMEM((CAP,), jnp.float32),
            pltpu.VMEM((CAP,), jnp.int32),
            pltpu.SemaphoreType.DMA,
        ],
    )
    def collect(bm_hbm, pb_hbm, thr_hbm, val_hbm, idx_hbm,
                bmbuf, bidbuf, pidbuf, vbuf, thrbuf, cbuf, ibuf, sem):
        wid = lax.axis_index("s") * 2 + lax.axis_index("c")
        base = wid * ROWS_PW
        pltpu.sync_copy(thr_hbm.at[pl.ds(base * 16, ROWS_PW * 16)], thrbuf)
        iot = lax.iota(jnp.int32, 16)
        neg = jnp.full((16,), -1.0, jnp.float32)
        zero = jnp.zeros((16,), jnp.int32)
        for k in range((QB + 16) // 16):
            bidbuf[pl.ds(k * 16, 16)] = zero
            pidbuf[pl.ds(k * 16, 16)] = zero

        def row_body(j, _):
            r = base + j
            pltpu.sync_copy(bm_hbm.at[r], bmbuf)
            tvec = thrbuf[pl.ds(j * 16, 16)]

            def bm_chunk(i, boff):
                v = bmbuf[pl.ds(i * 16, 16)]
                mask = v >= tvec
                cnt = plsc.all_reduce_population_count(mask)[0]
                lbid = iot + i * 16
                plsc.store_compressed(bidbuf.at[pl.ds(boff, 16)], lbid,
                                      mask=mask)
                plsc.store_compressed(pidbuf.at[pl.ds(boff, 16)],
                                      (lbid >> 2) + r * 128, mask=mask)
                return jnp.minimum(boff + cnt, QB)

            nq = lax.fori_loop(0, NBM // 16, bm_chunk, jnp.int32(0), unroll=4)

            pltpu.async_copy(pb_hbm.at[pidbuf.at[pl.ds(0, QB)]],
                             vbuf.at[...], sem).wait()

            for k in range(CAP // 16):
                cbuf[pl.ds(k * 16, 16)] = neg

            def blk(q, off):
                bv = bidbuf[pl.ds((q // 16) * 16, 16)]
                lbid = bv[jnp.broadcast_to(q % 16, (16,))]
                col0 = lbid * BMW + iot
                sub = (lbid & 3) * BMW
                rowsplat = jnp.broadcast_to(q, (16,)).astype(jnp.int32)

                def half(h, off):
                    v = plsc.load_gather(vbuf, [rowsplat, sub + h * 16 + iot])
                    mask = v >= tvec
                    cnt = plsc.all_reduce_population_count(mask)[0]
                    plsc.store_compressed(cbuf.at[pl.ds(off, 16)], v,
                                          mask=mask)
                    plsc.store_compressed(ibuf.at[pl.ds(off, 16)],
                                          col0 + h * 16, mask=mask)
                    return jnp.minimum(off + cnt, CAP - 16)

                return half(1, half(0, off))

            lax.fori_loop(0, nq, blk, jnp.int32(0))
            pltpu.sync_copy(cbuf, val_hbm.at[r])
            pltpu.sync_copy(ibuf, idx_hbm.at[r])
            return 0

        lax.fori_loop(0, ROWS_PW, row_body, 0)

    return collect


def _make_gather():
    mesh = plsc.VectorSubcoreMesh(core_axis_name="c", subcore_axis_name="s")
    npts = NPOINT * NSAMPLE

    @functools.partial(
        pl.kernel,
        out_type=jax.ShapeDtypeStruct((NJOB, npts), jnp.float32),
        mesh=mesh,
        compiler_params=pltpu.CompilerParams(needs_layout_passes=False),
        scratch_types=[
            pltpu.VMEM((N,), jnp.float32),
            pltpu.VMEM((npts,), jnp.int32),
            pltpu.VMEM((npts,), jnp.float32),
        ],
    )
    def gather(src_hbm, gidx_hbm, out_hbm, srcbuf, idxbuf, obuf):
        wid = lax.axis_index("s") * 2 + lax.axis_index("c")

        def do_job(job):
            b = job // NCH
            pltpu.sync_copy(src_hbm.at[job], srcbuf)
            pltpu.sync_copy(gidx_hbm.at[b], idxbuf)

            def chunk(i, _):
                idxv = idxbuf[pl.ds(i * 16, 16)]
                obuf[pl.ds(i * 16, 16)] = plsc.load_gather(srcbuf, [idxv])
                return 0

            lax.fori_loop(0, npts // 16, chunk, 0, unroll=8)
            pltpu.sync_copy(obuf, out_hbm.at[job])

        do_job(wid)

        @pl.when(wid < NJOB - NW)
        def _():
            do_job(wid + NW)

    return gather


def kernel(xyz, features, W1, g1, b1, m1, v1, W2, g2, b2, m2, v2, W3, bias3):
    xyz_trans, p = _scores(xyz, W1, g1, b1, m1, v1, W2, g2, b2, m2, v2, W3,
                           bias3)

    bm = _block_max(p)
    t = lax.top_k(bm, NSAMPLE)[0][:, NSAMPLE - 1]
    thr = jnp.broadcast_to(t[:, None], (NROW, 16)).reshape(NROW * 16)

    cval, cidx = _make_collect()(bm, p.reshape(NROW * 128, 128), thr)

    pos = lax.top_k(cval, NSAMPLE)[1]
    gidx = jnp.take_along_axis(cidx, pos, axis=1)
    gidx2 = gidx.reshape(B, NPOINT * NSAMPLE)

    src = jnp.concatenate([xyz_trans, features], axis=1)
    out = _make_gather()(src.reshape(NJOB, N), gidx2)
    return out.reshape(B, NCH, NPOINT, NSAMPLE)


# speedup vs baseline: 2.5100x; 2.5100x over previous
import functools

import jax
import jax.numpy as jnp
from jax import lax
from jax.experimental import pallas as pl
from jax.experimental.pallas import tpu as pltpu
from jax.experimental.pallas import tpu_sc as plsc

NPOINT = 512
NSAMPLE = 64
EPS = 1e-5
B = 2
N = 16384
C = 16
NROW = B * NPOINT
BLK = 4096
BMW = 32
NBM = N // BMW
CAP = 256
NW = 32
ROWS_PW = NROW // NW
NCH = 3 + C
NJOB = B * NCH


def _bn_eval(x, g, b, m, v):
    return (x - m[None, :, None]) / jnp.sqrt(v[None, :, None] + EPS) \
        * g[None, :, None] + b[None, :, None]


def _scores(xyz, W1, g1, b1, m1, v1, W2, g2, b2, m2, v2, W3, bias3):
    xyz_trans = jnp.transpose(xyz, (0, 2, 1))
    h = jnp.einsum('oi,bin->bon', W1, xyz_trans)
    h = jax.nn.relu(_bn_eval(h, g1, b1, m1, v1))
    h = jnp.einsum('oi,bin->bon', W2, h)
    h = jax.nn.relu(_bn_eval(h, g2, b2, m2, v2))
    logits = jnp.einsum('oi,bin->bon', W3, h) + bias3[None, :, None]
    return xyz_trans, jax.nn.sigmoid(logits)


def _bm_body(p_ref, bm_ref):
    bm_ref[0, 0] = jnp.max(
        p_ref[0].reshape(NPOINT, BLK // BMW, BMW), axis=-1)


def _block_max(p):
    nblk = N // BLK
    bm = pl.pallas_call(
        _bm_body,
        grid=(B, nblk),
        in_specs=[pl.BlockSpec((1, NPOINT, BLK), lambda b, j: (b, 0, j))],
        out_specs=pl.BlockSpec((1, 1, NPOINT, BLK // BMW),
                               lambda b, j: (b, j, 0, 0)),
        out_shape=jax.ShapeDtypeStruct((B, nblk, NPOINT, BLK // BMW),
                                       jnp.float32),
        compiler_params=pltpu.CompilerParams(
            dimension_semantics=("parallel", "parallel")),
    )(p)
    return bm.transpose(0, 2, 1, 3).reshape(NROW, NBM)


def _make_collect():
    mesh = plsc.VectorSubcoreMesh(core_axis_name="c", subcore_axis_name="s")
    QB = NSAMPLE + 16

    @functools.partial(
        pl.kernel,
        out_type=[
            jax.ShapeDtypeStruct((NROW, CAP), jnp.float32),
            jax.ShapeDtypeStruct((NROW, CAP), jnp.int32),
        ],
        mesh=mesh,
        compiler_params=pltpu.CompilerParams(needs_layout_passes=False),
        scratch_types=[
            pltpu.VMEM((2 * N,), jnp.float32),
            pltpu.VMEM((ROWS_PW * NBM,), jnp.float32),
            pltpu.VMEM((QB + 16,), jnp.int32),
            pltpu.VMEM((ROWS_PW * 16,), jnp.float32),
            pltpu.VMEM((CAP,), jnp.float32),
            pltpu.VMEM((CAP,), jnp.int32),
            pltpu.SemaphoreType.DMA,
            pltpu.SemaphoreType.DMA,
        ],
    )
    def collect(bm_hbm, p_hbm, thr_hbm, val_hbm, idx_hbm,
                rowbuf, bmbuf, bidbuf, thrbuf, cbuf, ibuf, sem0, sem1):
        wid = lax.axis_index("s") * 2 + lax.axis_index("c")
        base = wid * ROWS_PW
        pltpu.sync_copy(thr_hbm.at[pl.ds(base * 16, ROWS_PW * 16)], thrbuf)
        pltpu.sync_copy(bm_hbm.at[pl.ds(base * NBM, ROWS_PW * NBM)], bmbuf)
        iot = lax.iota(jnp.int32, 16)
        neg = jnp.full((16,), -1.0, jnp.float32)

        pltpu.async_copy(p_hbm.at[base], rowbuf.at[pl.ds(0, N)], sem0)

        def process(j, par):
            r = base + j
            tvec = thrbuf[pl.ds(j * 16, 16)]

            def bm_chunk(i, boff):
                v = bmbuf[pl.ds(j * NBM + i * 16, 16)]
                mask = v >= tvec
                cnt = plsc.all_reduce_population_count(mask)[0]
                plsc.store_compressed(bidbuf.at[pl.ds(boff, 16)], iot + i * 16,
                                      mask=mask)
                return jnp.minimum(boff + cnt, QB)

            nq = lax.fori_loop(0, NBM // 16, bm_chunk, jnp.int32(0), unroll=4)

            for k in range(CAP // 16):
                cbuf[pl.ds(k * 16, 16)] = neg

            pltpu.make_async_copy(p_hbm.at[r], rowbuf.at[pl.ds(par * N, N)],
                                  sem1 if par else sem0).wait()

            def blk(q, off):
                bv = bidbuf[pl.ds((q // 16) * 16, 16)]
                lbid = bv[jnp.broadcast_to(q % 16, (16,))]
                col0 = lbid * BMW + iot

                def half(h, off):
                    cols = col0 + h * 16
                    v = plsc.load_gather(rowbuf, [cols + par * N])
                    mask = v >= tvec
                    cnt = plsc.all_reduce_population_count(mask)[0]
                    plsc.store_compressed(cbuf.at[pl.ds(off, 16)], v,
                                          mask=mask)
                    plsc.store_compressed(ibuf.at[pl.ds(off, 16)], cols,
                                          mask=mask)
                    return jnp.minimum(off + cnt, CAP - 16)

                return half(1, half(0, off))

            lax.fori_loop(0, nq, blk, jnp.int32(0))
            pltpu.sync_copy(cbuf, val_hbm.at[r])
            pltpu.sync_copy(ibuf, idx_hbm.at[r])

        def pair_body(k, _):
            j0 = 2 * k
            pltpu.async_copy(p_hbm.at[base + j0 + 1], rowbuf.at[pl.ds(N, N)], sem1)
            process(j0, 0)

            @pl.when(j0 + 2 < ROWS_PW)
            def _():
                pltpu.async_copy(p_hbm.at[base + j0 + 2], rowbuf.at[pl.ds(0, N)], sem0)

            process(j0 + 1, 1)
            return 0

        lax.fori_loop(0, ROWS_PW // 2, pair_body, 0)

    return collect


def _make_gather():
    mesh = plsc.VectorSubcoreMesh(core_axis_name="c", subcore_axis_name="s")
    npts = NPOINT * NSAMPLE

    @functools.partial(
        pl.kernel,
        out_type=jax.ShapeDtypeStruct((NJOB, npts), jnp.float32),
        mesh=mesh,
        compiler_params=pltpu.CompilerParams(needs_layout_passes=False),
        scratch_types=[
            pltpu.VMEM((N,), jnp.float32),
            pltpu.VMEM((npts,), jnp.int32),
            pltpu.VMEM((npts,), jnp.float32),
        ],
    )
    def gather(src_hbm, gidx_hbm, out_hbm, srcbuf, idxbuf, obuf):
        wid = lax.axis_index("s") * 2 + lax.axis_index("c")

        def do_job(job):
            b = job // NCH
            pltpu.sync_copy(src_hbm.at[job], srcbuf)
            pltpu.sync_copy(gidx_hbm.at[b], idxbuf)

            def chunk(i, _):
                idxv = idxbuf[pl.ds(i * 16, 16)]
                obuf[pl.ds(i * 16, 16)] = plsc.load_gather(srcbuf, [idxv])
                return 0

            lax.fori_loop(0, npts // 16, chunk, 0, unroll=8)
            pltpu.sync_copy(obuf, out_hbm.at[job])

        do_job(wid)

        @pl.when(wid < NJOB - NW)
        def _():
            do_job(wid + NW)

    return gather


def kernel(xyz, features, W1, g1, b1, m1, v1, W2, g2, b2, m2, v2, W3, bias3):
    xyz_trans, p = _scores(xyz, W1, g1, b1, m1, v1, W2, g2, b2, m2, v2, W3,
                           bias3)

    bm = _block_max(p)
    t = lax.top_k(bm, NSAMPLE)[0][:, NSAMPLE - 1]
    thr = jnp.broadcast_to(t[:, None], (NROW, 16)).reshape(NROW * 16)

    cval, cidx = _make_collect()(bm.reshape(NROW * NBM), p.reshape(NROW, N), thr)

    pos = lax.top_k(cval, NSAMPLE)[1]
    gidx = jnp.take_along_axis(cidx, pos, axis=1)
    gidx2 = gidx.reshape(B, NPOINT * NSAMPLE)

    src = jnp.concatenate([xyz_trans, features], axis=1)
    out = _make_gather()(src.reshape(NJOB, N), gidx2)
    return out.reshape(B, NCH, NPOINT, NSAMPLE)


# speedup vs baseline: 2.5524x; 1.0169x over previous
import functools

import jax
import jax.numpy as jnp
from jax import lax
from jax.experimental import pallas as pl
from jax.experimental.pallas import tpu as pltpu
from jax.experimental.pallas import tpu_sc as plsc

NPOINT = 512
NSAMPLE = 64
EPS = 1e-5
B = 2
N = 16384
C = 16
NROW = B * NPOINT
BLK = 4096
BMW = 32
NBM = N // BMW
CAP = 256
NW = 32
ROWS_PW = NROW // NW
NCH = 3 + C
NJOB = B * NCH


def _bn_eval(x, g, b, m, v):
    return (x - m[None, :, None]) / jnp.sqrt(v[None, :, None] + EPS) \
        * g[None, :, None] + b[None, :, None]


def _scores(xyz, W1, g1, b1, m1, v1, W2, g2, b2, m2, v2, W3, bias3):
    xyz_trans = jnp.transpose(xyz, (0, 2, 1))
    h = jnp.einsum('oi,bin->bon', W1, xyz_trans)
    h = jax.nn.relu(_bn_eval(h, g1, b1, m1, v1))
    h = jnp.einsum('oi,bin->bon', W2, h)
    h = jax.nn.relu(_bn_eval(h, g2, b2, m2, v2))
    logits = jnp.einsum('oi,bin->bon', W3, h) + bias3[None, :, None]
    return xyz_trans, jax.nn.sigmoid(logits)


def _bm_body(p_ref, bm_ref):
    bm_ref[0, 0] = jnp.max(
        p_ref[0].reshape(NPOINT, BLK // BMW, BMW), axis=-1)


def _block_max(p):
    nblk = N // BLK
    bm = pl.pallas_call(
        _bm_body,
        grid=(B, nblk),
        in_specs=[pl.BlockSpec((1, NPOINT, BLK), lambda b, j: (b, 0, j))],
        out_specs=pl.BlockSpec((1, 1, NPOINT, BLK // BMW),
                               lambda b, j: (b, j, 0, 0)),
        out_shape=jax.ShapeDtypeStruct((B, nblk, NPOINT, BLK // BMW),
                                       jnp.float32),
        compiler_params=pltpu.CompilerParams(
            dimension_semantics=("parallel", "parallel")),
    )(p)
    return bm.transpose(0, 2, 1, 3).reshape(NROW, NBM)


def _make_collect():
    mesh = plsc.VectorSubcoreMesh(core_axis_name="c", subcore_axis_name="s")
    QB = NSAMPLE + 16

    @functools.partial(
        pl.kernel,
        out_type=[
            jax.ShapeDtypeStruct((NROW, CAP), jnp.float32),
            jax.ShapeDtypeStruct((NROW, CAP), jnp.int32),
        ],
        mesh=mesh,
        compiler_params=pltpu.CompilerParams(needs_layout_passes=False),
        scratch_types=[
            pltpu.VMEM((2 * N,), jnp.float32),
            pltpu.VMEM((ROWS_PW * NBM,), jnp.float32),
            pltpu.VMEM((QB + 16,), jnp.int32),
            pltpu.VMEM((ROWS_PW * 16,), jnp.float32),
            pltpu.VMEM((CAP,), jnp.float32),
            pltpu.VMEM((CAP,), jnp.int32),
            pltpu.SemaphoreType.DMA,
            pltpu.SemaphoreType.DMA,
        ],
    )
    def collect(bm_hbm, p_hbm, thr_hbm, val_hbm, idx_hbm,
                rowbuf, bmbuf, bidbuf, thrbuf, cbuf, ibuf, sem0, sem1):
        wid = lax.axis_index("s") * 2 + lax.axis_index("c")
        base = wid * ROWS_PW
        pltpu.sync_copy(thr_hbm.at[pl.ds(base * 16, ROWS_PW * 16)], thrbuf)
        pltpu.sync_copy(bm_hbm.at[pl.ds(base * NBM, ROWS_PW * NBM)], bmbuf)
        iot = lax.iota(jnp.int32, 16)
        neg = jnp.full((16,), -1.0, jnp.float32)

        pltpu.async_copy(p_hbm.at[base], rowbuf.at[pl.ds(0, N)], sem0)

        def process(j, par):
            r = base + j
            tvec = thrbuf[pl.ds(j * 16, 16)]

            def bm_chunk(i, boff):
                v = bmbuf[pl.ds(j * NBM + i * 16, 16)]
                mask = v >= tvec
                cnt = plsc.all_reduce_population_count(mask)[0]
                plsc.store_compressed(bidbuf.at[pl.ds(boff, 16)], iot + i * 16,
                                      mask=mask)
                return jnp.minimum(boff + cnt, QB)

            nq = lax.fori_loop(0, NBM // 16, bm_chunk, jnp.int32(0), unroll=4)

            for k in range(CAP // 16):
                cbuf[pl.ds(k * 16, 16)] = neg

            pltpu.make_async_copy(p_hbm.at[r], rowbuf.at[pl.ds(par * N, N)],
                                  sem1 if par else sem0).wait()

            def blk(q, off):
                bv = bidbuf[pl.ds((q // 16) * 16, 16)]
                lbid = bv[jnp.broadcast_to(q % 16, (16,))]
                col0 = lbid * BMW + iot

                def half(h, off):
                    cols = col0 + h * 16
                    v = plsc.load_gather(rowbuf, [cols + par * N])
                    mask = v >= tvec
                    cnt = plsc.all_reduce_population_count(mask)[0]
                    plsc.store_compressed(cbuf.at[pl.ds(off, 16)], v,
                                          mask=mask)
                    plsc.store_compressed(ibuf.at[pl.ds(off, 16)], cols,
                                          mask=mask)
                    return jnp.minimum(off + cnt, CAP - 16)

                return half(1, half(0, off))

            lax.fori_loop(0, nq, blk, jnp.int32(0))
            pltpu.sync_copy(cbuf, val_hbm.at[r])
            pltpu.sync_copy(ibuf, idx_hbm.at[r])

        def pair_body(k, _):
            j0 = 2 * k
            pltpu.async_copy(p_hbm.at[base + j0 + 1], rowbuf.at[pl.ds(N, N)], sem1)
            process(j0, 0)

            @pl.when(j0 + 2 < ROWS_PW)
            def _():
                pltpu.async_copy(p_hbm.at[base + j0 + 2], rowbuf.at[pl.ds(0, N)], sem0)

            process(j0 + 1, 1)
            return 0

        lax.fori_loop(0, ROWS_PW // 2, pair_body, 0)

    return collect


def _make_gather():
    mesh = plsc.VectorSubcoreMesh(core_axis_name="c", subcore_axis_name="s")
    npts = NPOINT * NSAMPLE
    HP = npts // 2
    NHALF = NJOB * 2

    @functools.partial(
        pl.kernel,
        out_type=jax.ShapeDtypeStruct((NJOB, npts), jnp.float32),
        mesh=mesh,
        compiler_params=pltpu.CompilerParams(needs_layout_passes=False),
        scratch_types=[
            pltpu.VMEM((N,), jnp.float32),
            pltpu.VMEM((HP,), jnp.int32),
            pltpu.VMEM((HP,), jnp.float32),
        ],
    )
    def gather(src_hbm, gidx_hbm, out_hbm, srcbuf, idxbuf, obuf):
        wid = lax.axis_index("s") * 2 + lax.axis_index("c")

        def do_half(h):
            job = h // 2
            part = h % 2
            b = job // NCH
            pltpu.sync_copy(src_hbm.at[job], srcbuf)
            pltpu.sync_copy(gidx_hbm.at[b, pl.ds(part * HP, HP)], idxbuf)

            def chunk(i, _):
                idxv = idxbuf[pl.ds(i * 16, 16)]
                obuf[pl.ds(i * 16, 16)] = plsc.load_gather(srcbuf, [idxv])
                return 0

            lax.fori_loop(0, HP // 16, chunk, 0, unroll=8)
            pltpu.sync_copy(obuf, out_hbm.at[job, pl.ds(part * HP, HP)])

        do_half(wid)
        do_half(wid + NW)

        @pl.when(wid < NHALF - 2 * NW)
        def _():
            do_half(wid + 2 * NW)

    return gather


def kernel(xyz, features, W1, g1, b1, m1, v1, W2, g2, b2, m2, v2, W3, bias3):
    xyz_trans, p = _scores(xyz, W1, g1, b1, m1, v1, W2, g2, b2, m2, v2, W3,
                           bias3)

    bm = _block_max(p)
    t = lax.top_k(bm, NSAMPLE)[0][:, NSAMPLE - 1]
    thr = jnp.broadcast_to(t[:, None], (NROW, 16)).reshape(NROW * 16)

    cval, cidx = _make_collect()(bm.reshape(NROW * NBM), p.reshape(NROW, N), thr)

    pos = lax.top_k(cval, NSAMPLE)[1]
    gidx = jnp.take_along_axis(cidx, pos, axis=1)
    gidx2 = gidx.reshape(B, NPOINT * NSAMPLE)

    src = jnp.concatenate([xyz_trans, features], axis=1)
    out = _make_gather()(src.reshape(NJOB, N), gidx2)
    return out.reshape(B, NCH, NPOINT, NSAMPLE)
